# baseline (speedup 1.0000x reference)
import jax
import jax.numpy as jnp
from jax import lax
from jax.experimental import pallas as pl
from jax.experimental.pallas import tpu as pltpu
from jax.experimental.pallas import tpu_sc as plsc

N_CLS = 1000
DIM = 768
N_CTX = 4
SUF = 72
PRE_W = DIM
CTX_W = N_CTX * DIM
SUF_W = SUF * DIM
ROW_W = PRE_W + CTX_W + SUF_W

NW = 32
BLK = 32
TAIL = N_CLS - BLK * (NW - 1)


def _row_copies(prefix_hbm, ctx_hbm, suffix_hbm, out_hbm, sem, row):
    o = pl.multiple_of(row * ROW_W, 8)
    po = pl.multiple_of(row * PRE_W, 8)
    so = pl.multiple_of(row * SUF_W, 8)
    return (
        pltpu.make_async_copy(
            prefix_hbm.at[pl.ds(po, PRE_W)], out_hbm.at[pl.ds(o, PRE_W)], sem
        ),
        pltpu.make_async_copy(
            ctx_hbm, out_hbm.at[pl.ds(o + PRE_W, CTX_W)], sem
        ),
        pltpu.make_async_copy(
            suffix_hbm.at[pl.ds(so, SUF_W)],
            out_hbm.at[pl.ds(o + PRE_W + CTX_W, SUF_W)],
            sem,
        ),
    )


def _sc_body(prefix_hbm, ctx_hbm, suffix_hbm, out_hbm, sem):
    c = lax.axis_index("c")
    s = lax.axis_index("s")
    wid = s * 2 + c
    base = wid * BLK
    nrows = jnp.where(wid == NW - 1, TAIL, BLK)

    def fire(j, _):
        for cp in _row_copies(prefix_hbm, ctx_hbm, suffix_hbm, out_hbm, sem, base + j):
            cp.start()
        return 0

    lax.fori_loop(0, nrows, fire, 0)

    def drain(j, _):
        for cp in _row_copies(prefix_hbm, ctx_hbm, suffix_hbm, out_hbm, sem, base + j):
            cp.wait()
        return 0

    lax.fori_loop(0, nrows, drain, 0)


def kernel(token_prefix, ctx, token_suffix):
    prefix1d = token_prefix.reshape(N_CLS * PRE_W)
    suffix1d = token_suffix.reshape(N_CLS * SUF_W)
    ctx1d = ctx.reshape(CTX_W)
    out1d = pl.kernel(
        _sc_body,
        out_type=jax.ShapeDtypeStruct((N_CLS * ROW_W,), jnp.float32),
        mesh=plsc.VectorSubcoreMesh(core_axis_name="c", subcore_axis_name="s"),
        scratch_types=[pltpu.SemaphoreType.DMA],
    )(prefix1d, ctx1d, suffix1d)
    return out1d.reshape(N_CLS, 1 + N_CTX + SUF, DIM)


# speedup vs baseline: 5.7504x; 5.7504x over previous
import jax
import jax.numpy as jnp
from jax import lax
from jax.experimental import pallas as pl
from jax.experimental.pallas import tpu as pltpu
from jax.experimental.pallas import tpu_sc as plsc

N_CLS = 1000
DIM = 768
N_CTX = 4
SUF = 72
PRE_W = DIM
CTX_W = N_CTX * DIM
SUF_W = SUF * DIM
ROW_W = PRE_W + CTX_W + SUF_W

NW = 32
BLK = 32


def _sc_body(prefix_hbm, ctx_hbm, suffix_hbm, out_hbm, buf0, buf1, fs0, fs1, ss0, ss1):
    c = lax.axis_index("c")
    s = lax.axis_index("s")
    wid = s * 2 + c
    base = jnp.minimum(wid * BLK, N_CLS - BLK)

    bufs = (buf0, buf1)
    fsem = (fs0, fs1)
    ssem = (ss0, ss1)

    pltpu.sync_copy(ctx_hbm, buf0.at[pl.ds(PRE_W, CTX_W)])
    pltpu.sync_copy(ctx_hbm, buf1.at[pl.ds(PRE_W, CTX_W)])

    def fills(j):
        row = base + j
        buf = bufs[j % 2]
        po = pl.multiple_of(row * PRE_W, 8)
        so = pl.multiple_of(row * SUF_W, 8)
        return (
            pltpu.make_async_copy(
                prefix_hbm.at[pl.ds(po, PRE_W)], buf.at[pl.ds(0, PRE_W)], fsem[j % 2]
            ),
            pltpu.make_async_copy(
                suffix_hbm.at[pl.ds(so, SUF_W)],
                buf.at[pl.ds(PRE_W + CTX_W, SUF_W)],
                fsem[j % 2],
            ),
        )

    def store(j):
        row = base + j
        o = pl.multiple_of(row * ROW_W, 8)
        return pltpu.make_async_copy(
            bufs[j % 2], out_hbm.at[pl.ds(o, ROW_W)], ssem[j % 2]
        )

    for cp in fills(0):
        cp.start()
    for j in range(BLK):
        for cp in fills(j):
            cp.wait()
        st = store(j)
        st.start()
        if j + 1 < BLK:
            if j >= 1:
                store(j - 1).wait()
            for cp in fills(j + 1):
                cp.start()
    store(BLK - 2).wait()
    store(BLK - 1).wait()


def kernel(token_prefix, ctx, token_suffix):
    prefix1d = token_prefix.reshape(N_CLS * PRE_W)
    suffix1d = token_suffix.reshape(N_CLS * SUF_W)
    ctx1d = ctx.reshape(CTX_W)
    out1d = pl.kernel(
        _sc_body,
        out_type=jax.ShapeDtypeStruct((N_CLS * ROW_W,), jnp.float32),
        mesh=plsc.VectorSubcoreMesh(core_axis_name="c", subcore_axis_name="s"),
        scratch_types=[
            pltpu.VMEM((ROW_W,), jnp.float32),
            pltpu.VMEM((ROW_W,), jnp.float32),
            pltpu.SemaphoreType.DMA,
            pltpu.SemaphoreType.DMA,
            pltpu.SemaphoreType.DMA,
            pltpu.SemaphoreType.DMA,
        ],
    )(prefix1d, ctx1d, suffix1d)
    return out1d.reshape(N_CLS, 1 + N_CTX + SUF, DIM)


# speedup vs baseline: 5.7515x; 1.0002x over previous
import jax
import jax.numpy as jnp
from jax import lax
from jax.experimental import pallas as pl
from jax.experimental.pallas import tpu as pltpu
from jax.experimental.pallas import tpu_sc as plsc

N_CLS = 1000
DIM = 768
N_CTX = 4
SUF = 72
ROWS = 1 + N_CTX + SUF

NW = 32
BLK = 32


def _sc_body(prefix_hbm, ctx_hbm, suffix_hbm, out_hbm, buf0, buf1, fs0, fs1, ss0, ss1):
    c = lax.axis_index("c")
    s = lax.axis_index("s")
    wid = s * 2 + c
    base = jnp.minimum(wid * BLK, N_CLS - BLK)

    bufs = (buf0, buf1)
    fsem = (fs0, fs1)
    ssem = (ss0, ss1)

    pltpu.sync_copy(ctx_hbm, buf0.at[pl.ds(1, N_CTX)])
    pltpu.sync_copy(ctx_hbm, buf1.at[pl.ds(1, N_CTX)])

    def fills(j):
        i = base + j
        buf = bufs[j % 2]
        return (
            pltpu.make_async_copy(
                prefix_hbm.at[i], buf.at[pl.ds(0, 1)], fsem[j % 2]
            ),
            pltpu.make_async_copy(
                suffix_hbm.at[i], buf.at[pl.ds(1 + N_CTX, SUF)], fsem[j % 2]
            ),
        )

    def store(j):
        return pltpu.make_async_copy(bufs[j % 2], out_hbm.at[base + j], ssem[j % 2])

    for cp in fills(0):
        cp.start()
    for j in range(BLK):
        for cp in fills(j):
            cp.wait()
        store(j).start()
        if j + 1 < BLK:
            if j >= 1:
                store(j - 1).wait()
            for cp in fills(j + 1):
                cp.start()
    store(BLK - 2).wait()
    store(BLK - 1).wait()


def kernel(token_prefix, ctx, token_suffix):
    return pl.kernel(
        _sc_body,
        out_type=jax.ShapeDtypeStruct((N_CLS, ROWS, DIM), jnp.float32),
        mesh=plsc.VectorSubcoreMesh(core_axis_name="c", subcore_axis_name="s"),
        compiler_params=pltpu.CompilerParams(use_tc_tiling_on_sc=False),
        scratch_types=[
            pltpu.VMEM((ROWS, DIM), jnp.float32),
            pltpu.VMEM((ROWS, DIM), jnp.float32),
            pltpu.SemaphoreType.DMA,
            pltpu.SemaphoreType.DMA,
            pltpu.SemaphoreType.DMA,
            pltpu.SemaphoreType.DMA,
        ],
    )(token_prefix, ctx, token_suffix)


# speedup vs baseline: 13.3649x; 2.3237x over previous
import jax
import jax.numpy as jnp
from jax import lax
from jax.experimental import pallas as pl
from jax.experimental.pallas import tpu as pltpu
from jax.experimental.pallas import tpu_sc as plsc

N_CLS = 1000
DIM = 768
N_CTX = 4
SUF = 72
ROWS = 1 + N_CTX + SUF
SHIFT = 1 + N_CTX
LANES = 16
NCOL = DIM // LANES

NW = 32
BLK = 32


def _copy_row(dst, dr, src, sr):
    for cc in range(NCOL):
        dst[dr, pl.ds(cc * LANES, LANES)] = src[sr, pl.ds(cc * LANES, LANES)]


def _sc_body(
    prefix_hbm, ctx_hbm, suffix_hbm, out_hbm,
    bufO0, bufO1, bufP0, bufP1, bufC, f0, f1, s0, s1,
):
    c = lax.axis_index("c")
    s = lax.axis_index("s")
    wid = s * 2 + c
    base = jnp.minimum(wid * BLK, N_CLS - BLK)

    bufO = (bufO0, bufO1)
    bufP = (bufP0, bufP1)
    fsem = (f0, f1)
    ssem = (s0, s1)

    pltpu.sync_copy(ctx_hbm, bufC)

    def gathers(j, b):
        i = base + j
        return (
            pltpu.make_async_copy(
                suffix_hbm.at[i], bufO[b].at[pl.ds(0, SUF)], fsem[b]
            ),
            pltpu.make_async_copy(prefix_hbm.at[i], bufP[b], fsem[b]),
        )

    def store(j, b):
        return pltpu.make_async_copy(bufO[b], out_hbm.at[base + j], ssem[b])

    def assemble(b):
        def sh(k, _):
            r = ROWS - 1 - k
            _copy_row(bufO[b], r, bufO[b], r - SHIFT)
            return 0

        lax.fori_loop(0, SUF, sh, 0)
        _copy_row(bufO[b], 0, bufP[b], 0)
        for r in range(N_CTX):
            _copy_row(bufO[b], 1 + r, bufC, r)

    def fire(cps):
        for cp in cps:
            cp.start()

    def drain(cps):
        for cp in cps:
            cp.wait()

    fire(gathers(0, 0))

    def body(k, _):
        j0 = 2 * k
        j1 = j0 + 1
        drain(gathers(j0, 0))

        @pl.when(k >= 1)
        def _():
            drain((store(j0 - 1, 1),))

        fire(gathers(j1, 1))
        assemble(0)
        fire((store(j0, 0),))
        drain(gathers(j1, 1))
        drain((store(j0, 0),))

        @pl.when(k < BLK // 2 - 1)
        def _():
            fire(gathers(j0 + 2, 0))

        assemble(1)
        fire((store(j1, 1),))
        return 0

    lax.fori_loop(0, BLK // 2, body, 0)
    drain((store(BLK - 1, 1),))


def kernel(token_prefix, ctx, token_suffix):
    return pl.kernel(
        _sc_body,
        out_type=jax.ShapeDtypeStruct((N_CLS, ROWS, DIM), jnp.float32),
        mesh=plsc.VectorSubcoreMesh(core_axis_name="c", subcore_axis_name="s"),
        scratch_types=[
            pltpu.VMEM((ROWS, DIM), jnp.float32),
            pltpu.VMEM((ROWS, DIM), jnp.float32),
            pltpu.VMEM((1, DIM), jnp.float32),
            pltpu.VMEM((1, DIM), jnp.float32),
            pltpu.VMEM((N_CTX, DIM), jnp.float32),
            pltpu.SemaphoreType.DMA,
            pltpu.SemaphoreType.DMA,
            pltpu.SemaphoreType.DMA,
            pltpu.SemaphoreType.DMA,
        ],
    )(token_prefix, ctx, token_suffix)


# speedup vs baseline: 24.3209x; 1.8198x over previous
import jax
import jax.numpy as jnp
from jax import lax
from jax.experimental import pallas as pl
from jax.experimental.pallas import tpu as pltpu
from jax.experimental.pallas import tpu_sc as plsc

N_CLS = 1000
DIM = 768
N_CTX = 4
SUF = 72
ROWS = 1 + N_CTX + SUF
SHIFT = 1 + N_CTX
LANES = 16
NCOL = DIM // LANES

NW = 32
BLK = 32


def _copy_row(dst, dr, src, sr):
    vals = [src[sr, pl.ds(cc * LANES, LANES)] for cc in range(NCOL)]
    for cc in range(NCOL):
        dst[dr, pl.ds(cc * LANES, LANES)] = vals[cc]


def _sc_body(
    prefix_hbm, ctx_hbm, suffix_hbm, out_hbm,
    bufO0, bufO1, bufP0, bufP1, bufC, f0, f1, s0, s1,
):
    c = lax.axis_index("c")
    s = lax.axis_index("s")
    wid = s * 2 + c
    base = jnp.minimum(wid * BLK, N_CLS - BLK)

    bufO = (bufO0, bufO1)
    bufP = (bufP0, bufP1)
    fsem = (f0, f1)
    ssem = (s0, s1)

    pltpu.sync_copy(ctx_hbm, bufC)

    def gathers(j, b):
        i = base + j
        return (
            pltpu.make_async_copy(
                suffix_hbm.at[i], bufO[b].at[pl.ds(0, SUF)], fsem[b]
            ),
            pltpu.make_async_copy(prefix_hbm.at[i], bufP[b], fsem[b]),
        )

    def store(j, b):
        return pltpu.make_async_copy(bufO[b], out_hbm.at[base + j], ssem[b])

    def assemble(b):
        @plsc.parallel_loop(0, SUF, unroll=2)
        def sh(k):
            r = ROWS - 1 - k
            _copy_row(bufO[b], r, bufO[b], r - SHIFT)
        _copy_row(bufO[b], 0, bufP[b], 0)
        for r in range(N_CTX):
            _copy_row(bufO[b], 1 + r, bufC, r)

    def fire(cps):
        for cp in cps:
            cp.start()

    def drain(cps):
        for cp in cps:
            cp.wait()

    fire(gathers(0, 0))

    def body(k, _):
        j0 = 2 * k
        j1 = j0 + 1
        drain(gathers(j0, 0))

        @pl.when(k >= 1)
        def _():
            drain((store(j0 - 1, 1),))

        fire(gathers(j1, 1))
        assemble(0)
        fire((store(j0, 0),))
        drain(gathers(j1, 1))
        drain((store(j0, 0),))

        @pl.when(k < BLK // 2 - 1)
        def _():
            fire(gathers(j0 + 2, 0))

        assemble(1)
        fire((store(j1, 1),))
        return 0

    lax.fori_loop(0, BLK // 2, body, 0)
    drain((store(BLK - 1, 1),))


def kernel(token_prefix, ctx, token_suffix):
    return pl.kernel(
        _sc_body,
        out_type=jax.ShapeDtypeStruct((N_CLS, ROWS, DIM), jnp.float32),
        mesh=plsc.VectorSubcoreMesh(core_axis_name="c", subcore_axis_name="s"),
        scratch_types=[
            pltpu.VMEM((ROWS, DIM), jnp.float32),
            pltpu.VMEM((ROWS, DIM), jnp.float32),
            pltpu.VMEM((1, DIM), jnp.float32),
            pltpu.VMEM((1, DIM), jnp.float32),
            pltpu.VMEM((N_CTX, DIM), jnp.float32),
            pltpu.SemaphoreType.DMA,
            pltpu.SemaphoreType.DMA,
            pltpu.SemaphoreType.DMA,
            pltpu.SemaphoreType.DMA,
        ],
    )(token_prefix, ctx, token_suffix)
